# baseline (speedup 1.0000x reference)
import jax
import jax.numpy as jnp
from jax.experimental import pallas as pl
from jax.experimental.pallas import tpu as pltpu
from jax.experimental.pallas import tpu_sc as plsc

B, L, HIDDEN = 2, 2048, 1024
HEADS, HEAD_DIM = 16, 64
CLUSTERS, ITERS = 128, 2

_f32 = jnp.float32
_NIDX = B * HEADS * L
_NROWS = B * HEADS * CLUSTERS
_GW = 256
_TW = 128
_LT = 1024


def _mm_seq(a, w, chunk=256):
    k = a.shape[1]
    acc = jnp.dot(a[:, 0:chunk], w[0:chunk], preferred_element_type=_f32)
    for i in range(1, k // chunk):
        acc = acc + jnp.dot(a[:, chunk * i:chunk * (i + 1)],
                            w[chunk * i:chunk * (i + 1)],
                            preferred_element_type=_f32)
    return acc


def _mm_t_seq(a, bmat, chunk=256):
    k = a.shape[0]
    dn = (((0,), (0,)), ((), ()))
    acc = jax.lax.dot_general(a[0:chunk], bmat[0:chunk], dn,
                              preferred_element_type=_f32)
    for i in range(1, k // chunk):
        acc = acc + jax.lax.dot_general(a[chunk * i:chunk * (i + 1)],
                                        bmat[chunk * i:chunk * (i + 1)], dn,
                                        preferred_element_type=_f32)
    return acc


def _attn_body(x_ref, xi_ref, maskc_ref,
               wqkv_ref, bqkv_ref, outc_ref, gid_ref):
    h = pl.program_id(0)
    x = x_ref[0]
    w = wqkv_ref[0]
    bias = bqkv_ref[0]
    qkv = _mm_seq(x, w) + bias
    q = qkv[:, 0:HEAD_DIM]
    k = qkv[:, HEAD_DIM:2 * HEAD_DIM]
    v = qkv[:, 2 * HEAD_DIM:3 * HEAD_DIM]
    maskc = maskc_ref[0]

    cent = (_mm_seq(xi_ref[0], w) + bias)[:, 0:HEAD_DIM]
    qsq = jnp.sum(q * q, axis=1, keepdims=True)
    iota_c = jax.lax.broadcasted_iota(jnp.int32, (L, CLUSTERS), 1)
    ones_col = jnp.ones((L, 1), _f32)
    grp = None
    for _ in range(ITERS):
        centsq = jnp.sum(cent * cent, axis=1)
        qc = jax.lax.dot_general(q, cent, (((1,), (1,)), ((), ())),
                                 preferred_element_type=_f32)
        d = qsq - 2.0 * qc + centsq.reshape(1, CLUSTERS)
        dmin = jnp.min(d, axis=1, keepdims=True)
        grp = jnp.min(jnp.where(d == dmin, iota_c, CLUSTERS),
                      axis=1, keepdims=True)
        onehot = jnp.where(iota_c == grp, 1.0, 0.0).astype(_f32) * maskc
        counts = jax.lax.dot_general(onehot, ones_col, (((0,), (0,)), ((), ())),
                                     preferred_element_type=_f32)
        sums = _mm_t_seq(onehot, q)
        new_cent = sums / jnp.maximum(counts, 1.0)
        cent = jnp.where(counts > 0, new_cent, cent)

    scale = _f32(1.0 / (HEAD_DIM ** 0.5))
    logits_t = jax.lax.dot_general(k, cent, (((1,), (1,)), ((), ())),
                                   preferred_element_type=_f32) * scale
    logits_t = jnp.where(maskc > 0.0, logits_t, _f32(-1e9))
    mx = jnp.max(logits_t, axis=0, keepdims=True)
    e = jnp.exp(logits_t - mx)
    a = e / jnp.sum(e, axis=0, keepdims=True)
    out_c = jax.lax.dot_general(a, v, (((0,), (0,)), ((), ())),
                                preferred_element_type=_f32)
    outc_ref[0] = jnp.concatenate(
        [out_c, jnp.zeros((CLUSTERS, _TW - HEAD_DIM), _f32)],
        axis=1)
    gid_ref[0] = grp + h * CLUSTERS


def _proj_body(g_ref, maskc_ref, wo_ref, bo_ref, y_ref):
    g = g_ref[0]
    y = jnp.dot(g, wo_ref[...], preferred_element_type=_f32)
    y_ref[0] = y * maskc_ref[0] + bo_ref[...]


def _sc_gather(table, idx, nidx):
    mesh = plsc.VectorSubcoreMesh(core_axis_name="core",
                                  subcore_axis_name="subcore")

    @pl.kernel(out_type=jax.ShapeDtypeStruct((nidx, _TW), _f32),
               mesh=mesh)
    def gather_kernel(tab_hbm, i_hbm, o_hbm):
        def body(i_vmem, o_vmem):
            pltpu.sync_copy(tab_hbm.at[i_vmem.at[0]], o_vmem)

        pltpu.emit_pipeline(
            body,
            grid=(nidx // _GW,),
            in_specs=[pl.BlockSpec((1, _GW), index_map=lambda i: (0, i))],
            out_specs=[pl.BlockSpec((_GW, _TW), index_map=lambda i: (i, 0))],
            core_axis_name=("core", "subcore"),
            dimension_semantics=(pltpu.PARALLEL,),
        )(i_hbm, o_hbm)

    return gather_kernel(table, idx)


def kernel(X, attn_mask, length_mask, Wq, bq, Wk, bk, Wv, bv, Wo, bo):
    pos = jnp.arange(L, dtype=jnp.int32)
    maskf = (attn_mask & (pos[None, :] < length_mask[:, None])).astype(_f32)
    maskc = maskf.reshape(B, L, 1)
    init_idx = jnp.linspace(0, L - 1, CLUSTERS).astype(jnp.int32)
    xinit = X[:, init_idx, :]

    def _hm(wmat):
        return wmat.reshape(HIDDEN, HEADS, HEAD_DIM).transpose(1, 0, 2)
    wqkv3 = jnp.concatenate([_hm(Wq), _hm(Wk), _hm(Wv)], axis=2)
    bqkv3 = jnp.concatenate([bq.reshape(HEADS, 1, HEAD_DIM),
                             bk.reshape(HEADS, 1, HEAD_DIM),
                             bv.reshape(HEADS, 1, HEAD_DIM)], axis=2)
    wo_pad = jnp.pad(Wo.reshape(HEADS, HEAD_DIM, HIDDEN),
                     ((0, 0), (0, _TW - HEAD_DIM), (0, 0)))
    wo_big = wo_pad.reshape(HEADS * _TW, HIDDEN)

    ys = []
    for b in range(B):
        out_c, gid = pl.pallas_call(
            _attn_body,
            grid=(HEADS,),
            in_specs=[
                pl.BlockSpec((1, L, HIDDEN), lambda h: (0, 0, 0)),
                pl.BlockSpec((1, CLUSTERS, HIDDEN), lambda h: (0, 0, 0)),
                pl.BlockSpec((1, L, 1), lambda h: (0, 0, 0)),
                pl.BlockSpec((1, HIDDEN, 3 * HEAD_DIM), lambda h: (h, 0, 0)),
                pl.BlockSpec((1, 1, 3 * HEAD_DIM), lambda h: (h, 0, 0)),
            ],
            out_specs=[
                pl.BlockSpec((1, CLUSTERS, _TW), lambda h: (h, 0, 0)),
                pl.BlockSpec((1, L, 1), lambda h: (h, 0, 0)),
            ],
            out_shape=[
                jax.ShapeDtypeStruct((HEADS, CLUSTERS, _TW), _f32),
                jax.ShapeDtypeStruct((HEADS, L, 1), jnp.int32),
            ],
        )(X[b:b + 1], xinit[b:b + 1], maskc[b:b + 1], wqkv3, bqkv3)

        table = out_c.reshape(HEADS * CLUSTERS, _TW)
        idxp = gid.reshape(HEADS, L).transpose(1, 0).reshape(1, HEADS * L)
        gath = _sc_gather(table, idxp, HEADS * L).reshape(1, L, HEADS * _TW)

        y = pl.pallas_call(
            _proj_body,
            grid=(1, L // _LT),
            in_specs=[
                pl.BlockSpec((1, _LT, HEADS * _TW), lambda b2, t: (0, t, 0)),
                pl.BlockSpec((1, _LT, 1), lambda b2, t: (0, t, 0)),
                pl.BlockSpec((HEADS * _TW, HIDDEN), lambda b2, t: (0, 0)),
                pl.BlockSpec((1, HIDDEN), lambda b2, t: (0, 0)),
            ],
            out_specs=pl.BlockSpec((1, _LT, HIDDEN), lambda b2, t: (0, t, 0)),
            out_shape=jax.ShapeDtypeStruct((1, L, HIDDEN), _f32),
        )(gath, maskc[b:b + 1], wo_big, bo.reshape(1, -1))
        ys.append(y)
    return jnp.concatenate(ys, axis=0)


# speedup vs baseline: 1.1143x; 1.1143x over previous
import jax
import jax.numpy as jnp
from jax.experimental import pallas as pl
from jax.experimental.pallas import tpu as pltpu
from jax.experimental.pallas import tpu_sc as plsc

B, L, HIDDEN = 2, 2048, 1024
HEADS, HEAD_DIM = 16, 64
CLUSTERS, ITERS = 128, 2

_f32 = jnp.float32
_NIDX = B * HEADS * L
_NROWS = B * HEADS * CLUSTERS
_GW = 256
_TW = 128
_LT = 1024


def _mm_seq(a, w, chunk=256):
    k = a.shape[1]
    acc = jnp.dot(a[:, 0:chunk], w[0:chunk], preferred_element_type=_f32)
    for i in range(1, k // chunk):
        acc = acc + jnp.dot(a[:, chunk * i:chunk * (i + 1)],
                            w[chunk * i:chunk * (i + 1)],
                            preferred_element_type=_f32)
    return acc


def _mm_t_seq(a, bmat, chunk=256):
    k = a.shape[0]
    dn = (((0,), (0,)), ((), ()))
    acc = jax.lax.dot_general(a[0:chunk], bmat[0:chunk], dn,
                              preferred_element_type=_f32)
    for i in range(1, k // chunk):
        acc = acc + jax.lax.dot_general(a[chunk * i:chunk * (i + 1)],
                                        bmat[chunk * i:chunk * (i + 1)], dn,
                                        preferred_element_type=_f32)
    return acc


def _attn_body(x_ref, xi_ref, maskc_ref,
               wqkv_ref, bqkv_ref, outc_ref, gid_ref):
    b = pl.program_id(0)
    h = pl.program_id(1)
    x = x_ref[0]
    w = wqkv_ref[0]
    bias = bqkv_ref[0]
    qkv = _mm_seq(x, w) + bias
    q = qkv[:, 0:HEAD_DIM]
    k = qkv[:, HEAD_DIM:2 * HEAD_DIM]
    v = qkv[:, 2 * HEAD_DIM:3 * HEAD_DIM]
    maskc = maskc_ref[0]

    cent = (_mm_seq(xi_ref[0], w) + bias)[:, 0:HEAD_DIM]
    qsq = jnp.sum(q * q, axis=1, keepdims=True)
    iota_c = jax.lax.broadcasted_iota(jnp.int32, (L, CLUSTERS), 1)
    ones_col = jnp.ones((L, 1), _f32)
    grp = None
    for _ in range(ITERS):
        centsq = jnp.sum(cent * cent, axis=1)
        qc = jax.lax.dot_general(q, cent, (((1,), (1,)), ((), ())),
                                 preferred_element_type=_f32)
        d = qsq - 2.0 * qc + centsq.reshape(1, CLUSTERS)
        dmin = jnp.min(d, axis=1, keepdims=True)
        grp = jnp.min(jnp.where(d == dmin, iota_c, CLUSTERS),
                      axis=1, keepdims=True)
        onehot = jnp.where(iota_c == grp, 1.0, 0.0).astype(_f32) * maskc
        counts = jax.lax.dot_general(onehot, ones_col, (((0,), (0,)), ((), ())),
                                     preferred_element_type=_f32)
        sums = _mm_t_seq(onehot, q)
        new_cent = sums / jnp.maximum(counts, 1.0)
        cent = jnp.where(counts > 0, new_cent, cent)

    scale = _f32(1.0 / (HEAD_DIM ** 0.5))
    logits_t = jax.lax.dot_general(k, cent, (((1,), (1,)), ((), ())),
                                   preferred_element_type=_f32) * scale
    logits_t = jnp.where(maskc > 0.0, logits_t, _f32(-1e9))
    mx = jnp.max(logits_t, axis=0, keepdims=True)
    e = jnp.exp(logits_t - mx)
    a = e / jnp.sum(e, axis=0, keepdims=True)
    out_c = jax.lax.dot_general(a, v, (((0,), (0,)), ((), ())),
                                preferred_element_type=_f32)
    outc_ref[0, 0] = jnp.concatenate(
        [out_c, jnp.zeros((CLUSTERS, _TW - HEAD_DIM), _f32)],
        axis=1)
    gid_ref[0, 0] = jnp.transpose(grp + (b * HEADS + h) * CLUSTERS,
                                  (1, 0))


def _proj_body(g_ref, maskc_ref, wo_ref, bo_ref, y_ref):
    g = g_ref[0]
    y = jnp.dot(g, wo_ref[...], preferred_element_type=_f32)
    y_ref[0] = y * maskc_ref[0] + bo_ref[...]


def _sc_gather(table, idx, nidx):
    mesh = plsc.VectorSubcoreMesh(core_axis_name="core",
                                  subcore_axis_name="subcore")

    @pl.kernel(out_type=jax.ShapeDtypeStruct((nidx, _TW), _f32),
               mesh=mesh)
    def gather_kernel(tab_hbm, i_hbm, o_hbm):
        def body(i_vmem, o_vmem):
            pltpu.sync_copy(tab_hbm.at[i_vmem.at[0]], o_vmem)

        pltpu.emit_pipeline(
            body,
            grid=(nidx // _GW,),
            in_specs=[pl.BlockSpec((1, _GW), index_map=lambda i: (0, i))],
            out_specs=[pl.BlockSpec((_GW, _TW), index_map=lambda i: (i, 0))],
            core_axis_name=("core", "subcore"),
            dimension_semantics=(pltpu.PARALLEL,),
        )(i_hbm, o_hbm)

    return gather_kernel(table, idx)


def kernel(X, attn_mask, length_mask, Wq, bq, Wk, bk, Wv, bv, Wo, bo):
    pos = jnp.arange(L, dtype=jnp.int32)
    maskf = (attn_mask & (pos[None, :] < length_mask[:, None])).astype(_f32)
    maskc = maskf.reshape(B, L, 1)
    init_idx = jnp.linspace(0, L - 1, CLUSTERS).astype(jnp.int32)
    xinit = X[:, init_idx, :]

    def _hm(wmat):
        return wmat.reshape(HIDDEN, HEADS, HEAD_DIM).transpose(1, 0, 2)
    wqkv3 = jnp.concatenate([_hm(Wq), _hm(Wk), _hm(Wv)], axis=2)
    bqkv3 = jnp.concatenate([bq.reshape(HEADS, 1, HEAD_DIM),
                             bk.reshape(HEADS, 1, HEAD_DIM),
                             bv.reshape(HEADS, 1, HEAD_DIM)], axis=2)
    wo_pad = jnp.pad(Wo.reshape(HEADS, HEAD_DIM, HIDDEN),
                     ((0, 0), (0, _TW - HEAD_DIM), (0, 0)))
    wo_big = wo_pad.reshape(HEADS * _TW, HIDDEN)

    out_c, gid = pl.pallas_call(
        _attn_body,
        grid=(B, HEADS),
        in_specs=[
            pl.BlockSpec((1, L, HIDDEN), lambda b, h: (b, 0, 0)),
            pl.BlockSpec((1, CLUSTERS, HIDDEN), lambda b, h: (b, 0, 0)),
            pl.BlockSpec((1, L, 1), lambda b, h: (b, 0, 0)),
            pl.BlockSpec((1, HIDDEN, 3 * HEAD_DIM), lambda b, h: (h, 0, 0)),
            pl.BlockSpec((1, 1, 3 * HEAD_DIM), lambda b, h: (h, 0, 0)),
        ],
        out_specs=[
            pl.BlockSpec((1, 1, CLUSTERS, _TW), lambda b, h: (b, h, 0, 0)),
            pl.BlockSpec((1, 1, 1, L), lambda b, h: (b, h, 0, 0)),
        ],
        out_shape=[
            jax.ShapeDtypeStruct((B, HEADS, CLUSTERS, _TW), _f32),
            jax.ShapeDtypeStruct((B, HEADS, 1, L), jnp.int32),
        ],
    )(X, xinit, maskc, wqkv3, bqkv3)

    table = out_c.reshape(_NROWS, _TW)
    idxp = gid.reshape(B, HEADS, L).transpose(0, 2, 1).reshape(1, _NIDX)
    gath = _sc_gather(table, idxp, _NIDX).reshape(B, L, HEADS * _TW)

    out = pl.pallas_call(
        _proj_body,
        grid=(B, L // _LT),
        in_specs=[
            pl.BlockSpec((1, _LT, HEADS * _TW), lambda b, t: (b, t, 0)),
            pl.BlockSpec((1, _LT, 1), lambda b, t: (b, t, 0)),
            pl.BlockSpec((HEADS * _TW, HIDDEN), lambda b, t: (0, 0)),
            pl.BlockSpec((1, HIDDEN), lambda b, t: (0, 0)),
        ],
        out_specs=pl.BlockSpec((1, _LT, HIDDEN), lambda b, t: (b, t, 0)),
        out_shape=jax.ShapeDtypeStruct((B, L, HIDDEN), _f32),
    )(gath, maskc, wo_big, bo.reshape(1, -1))
    return out
